# baseline (speedup 1.0000x reference)
import functools

import jax
import jax.numpy as jnp
from jax import lax
from jax.experimental import pallas as pl
from jax.experimental.pallas import tpu as pltpu
from jax.experimental.pallas import tpu_sc as plsc

_BS = 8
_N = 10000
_SEQ = 9
_C = 128
_M = 2500
_MP = 2560
_NNZ = 7500

_NC = 2
_NS = 16
_NP = 4
_KT = 480
_NNZP = _NS * _KT
_G = 8
_NG = _KT // _G
_RW = 80


def _mm_body(x_ref, w_ref, o_ref):
    o_ref[...] = jnp.dot(x_ref[...], w_ref[...],
                         preferred_element_type=jnp.float32)


def _matmul_tc(xr, wt):
    rows = xr.shape[0]
    blk = 800
    grid = rows // blk
    return pl.pallas_call(
        _mm_body,
        grid=(grid,),
        in_specs=[
            pl.BlockSpec((blk, _C), lambda i: (i, 0)),
            pl.BlockSpec((_C, _SEQ * _C), lambda i: (0, 0)),
        ],
        out_specs=pl.BlockSpec((blk, _SEQ * _C), lambda i: (i, 0)),
        out_shape=jax.ShapeDtypeStruct((rows, _SEQ * _C), jnp.float32),
    )(xr, wt)


def _sc_body(z2_hbm, gidx_hbm, rows_hbm, vals_hbm, batoff_hbm,
             bias_hbm, out_hbm,
             gidx_v, rows_v, vals_v, boff_v, zbuf, zbuf2, stage, stage2,
             bias_v, zerov, acc, sem, sem2, sem3, sem4):
    cid = lax.axis_index("c")
    sid = lax.axis_index("s")

    pltpu.sync_copy(gidx_hbm.at[sid], gidx_v)
    pltpu.sync_copy(rows_hbm.at[sid], rows_v)
    pltpu.sync_copy(vals_hbm.at[sid], vals_v)
    pltpu.sync_copy(batoff_hbm.at[cid], boff_v)
    pltpu.sync_copy(bias_hbm, bias_v)

    for i in range(16):
        for cc in range(_C // 16):
            zerov[i, pl.ds(cc * 16, 16)] = jnp.zeros((16,), jnp.float32)
    for jr in range(_G, 16):
        for cc in range(_C // 16):
            stage[jr, pl.ds(cc * 16, 16)] = jnp.zeros((16,), jnp.float32)
            stage2[jr, pl.ds(cc * 16, 16)] = jnp.zeros((16,), jnp.float32)

    boff = boff_v[0, :]

    def _off_row(i, carry):
        for c5 in range(_RW // 16):
            sl = pl.ds(c5 * 16, 16)
            gidx_v[i, sl] = gidx_v[i, sl] + boff
        return carry

    lax.fori_loop(0, _NG, _off_row, 0)
    delta = jnp.full((16,), _N * _SEQ, dtype=jnp.int32)

    def _do(g, zb, st):
        for j in range(_G):
            vrow = vals_v[g, pl.ds(j * 16, 16)]
            for cc in range(_C // 16):
                sl = pl.ds(cc * 16, 16)
                v = zb[j * _SEQ, sl]
                for sp in range(1, _SEQ):
                    v = v + zb[j * _SEQ + sp, sl]
                v = v + bias_v[cc, :]
                v = jnp.where(v > 0.0, v, jnp.exp(v) - 1.0)
                v = v * vrow
                st[j, sl] = v

    def _pair(i, carry):
        g0 = 2 * i
        g1 = g0 + 1
        h0 = pltpu.async_copy(z2_hbm.at[gidx_v.at[g0]], zbuf, sem)
        h1 = pltpu.async_copy(z2_hbm.at[gidx_v.at[g1]], zbuf2, sem3)
        h0.wait()
        _do(g0, zbuf, stage)
        s0 = pltpu.async_copy(stage, acc.at[rows_v.at[g0]], sem2, add=True)
        h1.wait()
        _do(g1, zbuf2, stage2)
        s1 = pltpu.async_copy(stage2, acc.at[rows_v.at[g1]], sem4, add=True)
        s0.wait()
        s1.wait()
        return carry

    for p in range(_NP):
        for r in range(10):
            pltpu.sync_copy(zerov, acc.at[pl.ds(sid * 160 + r * 16, 16)])
        plsc.subcore_barrier()
        lax.fori_loop(0, _NG // 2, _pair, 0)
        plsc.subcore_barrier()
        pltpu.sync_copy(
            acc.at[pl.ds(sid * 160, 160)],
            out_hbm.at[cid * _NP + p, pl.ds(sid * 160, 160)])
        if p + 1 < _NP:
            def _shift_row(i, carry):
                for c5 in range(_RW // 16):
                    sl = pl.ds(c5 * 16, 16)
                    gidx_v[i, sl] = gidx_v[i, sl] + delta
                return carry

            lax.fori_loop(0, _NG, _shift_row, 0)


_sc_fused = functools.partial(
    pl.kernel,
    out_type=jax.ShapeDtypeStruct((_BS, _MP, _C), jnp.float32),
    mesh=plsc.VectorSubcoreMesh(core_axis_name="c", subcore_axis_name="s",
                                num_cores=_NC, num_subcores=_NS),
    scratch_types=[
        pltpu.VMEM((_NG, _RW), jnp.int32),
        pltpu.VMEM((_NG, 16), jnp.int32),
        pltpu.VMEM((_NG, 128), jnp.float32),
        pltpu.VMEM((1, 16), jnp.int32),
        pltpu.VMEM((_RW, _C), jnp.float32),
        pltpu.VMEM((_RW, _C), jnp.float32),
        pltpu.VMEM((16, _C), jnp.float32),
        pltpu.VMEM((16, _C), jnp.float32),
        pltpu.VMEM((_C // 16, 16), jnp.float32),
        pltpu.VMEM((16, _C), jnp.float32),
        pltpu.VMEM_SHARED((_MP, _C), jnp.float32),
        pltpu.SemaphoreType.DMA,
        pltpu.SemaphoreType.DMA,
        pltpu.SemaphoreType.DMA,
        pltpu.SemaphoreType.DMA,
    ],
)(_sc_body)


def kernel(x, indices, dt_rows, dt_cols, dt_vals, W, b):
    idx32 = indices.astype(jnp.int32)
    cols = dt_cols.astype(jnp.int32)
    rows = dt_rows.astype(jnp.int32)

    pad = _NNZP - _NNZ
    cols_p = jnp.pad(cols, (0, pad))
    rows_p = jnp.pad(rows, (0, pad))
    vals_p = jnp.pad(dt_vals, (0, pad))

    nbr = jnp.take(idx32, cols_p, axis=0)
    g9 = nbr * _SEQ + jnp.arange(_SEQ, dtype=jnp.int32)[None, :]
    g72 = g9.reshape(_NS, _NG, _G * _SEQ)
    padcol = (jnp.arange(_NG, dtype=jnp.int32)[:, None] * 8
              + jnp.arange(_RW - _G * _SEQ, dtype=jnp.int32)[None, :]) % (
                  _N * _SEQ)
    gidx16 = jnp.concatenate(
        [g72, jnp.broadcast_to(padcol[None], (_NS, _NG, _RW - _G * _SEQ))],
        axis=2)
    rows16 = jnp.pad(rows_p.reshape(_NS, _NG, _G),
                     ((0, 0), (0, 0), (0, 16 - _G)))
    vals16 = jnp.broadcast_to(vals_p.reshape(_NS, _NG, _G, 1),
                              (_NS, _NG, _G, 16)).reshape(_NS, _NG, 128)

    batoff = jnp.broadcast_to(
        (jnp.arange(_NC, dtype=jnp.int32) * (_NP * _N * _SEQ))[:, None, None],
        (_NC, 1, 16))

    bias_b = b.reshape(_C // 16, 16).astype(jnp.float32)

    wt = W.reshape(_C, _SEQ, _C).transpose(2, 1, 0).reshape(_C, _SEQ * _C)
    z = _matmul_tc(x.reshape(_BS * _N, _C), wt.astype(jnp.float32))
    z2 = z.reshape(_BS * _N * _SEQ, _C)

    pooled_p = _sc_fused(z2, gidx16, rows16, vals16, batoff, bias_b)
    return pooled_p[:, :_M, :]


# speedup vs baseline: 1.0276x; 1.0276x over previous
import functools

import jax
import jax.numpy as jnp
from jax import lax
from jax.experimental import pallas as pl
from jax.experimental.pallas import tpu as pltpu
from jax.experimental.pallas import tpu_sc as plsc

_BS = 8
_N = 10000
_SEQ = 9
_C = 128
_M = 2500
_MP = 2560
_NNZ = 7500

_NC = 2
_NS = 16
_NP = 4
_KT = 472
_NNZP = _NS * _KT
_G = 8
_NG = _KT // _G
_RW = 80


def _mm_body(x_ref, w_ref, o_ref):
    o_ref[...] = jnp.dot(x_ref[...], w_ref[...],
                         preferred_element_type=jnp.float32)


def _matmul_tc(xr, wt):
    rows = xr.shape[0]
    blk = 800
    grid = rows // blk
    return pl.pallas_call(
        _mm_body,
        grid=(grid,),
        in_specs=[
            pl.BlockSpec((blk, _C), lambda i: (i, 0)),
            pl.BlockSpec((_C, _SEQ * _C), lambda i: (0, 0)),
        ],
        out_specs=pl.BlockSpec((blk, _SEQ * _C), lambda i: (i, 0)),
        out_shape=jax.ShapeDtypeStruct((rows, _SEQ * _C), jnp.float32),
    )(xr, wt)


def _sc_body(z2_hbm, gidx_hbm, rows_hbm, vals_hbm, batoff_hbm,
             bias_hbm, out_hbm,
             gidx_v, rows_v, vals_v, boff_v, zbuf, stage,
             bias_v, zerov, acc, sem, sem2):
    cid = lax.axis_index("c")
    sid = lax.axis_index("s")

    pltpu.sync_copy(gidx_hbm.at[sid], gidx_v)
    pltpu.sync_copy(rows_hbm.at[sid], rows_v)
    pltpu.sync_copy(vals_hbm.at[sid], vals_v)
    pltpu.sync_copy(batoff_hbm.at[cid], boff_v)
    pltpu.sync_copy(bias_hbm, bias_v)

    for i in range(16):
        for cc in range(_C // 16):
            zerov[i, pl.ds(cc * 16, 16)] = jnp.zeros((16,), jnp.float32)
    for jr in range(_G, 16):
        for cc in range(_C // 16):
            stage[jr, pl.ds(cc * 16, 16)] = jnp.zeros((16,), jnp.float32)

    boff = boff_v[0, :]

    def _off_row(i, carry):
        for c5 in range(_RW // 16):
            sl = pl.ds(c5 * 16, 16)
            gidx_v[i, sl] = gidx_v[i, sl] + boff
        return carry

    lax.fori_loop(0, _NG, _off_row, 0)
    delta = jnp.full((16,), _N * _SEQ, dtype=jnp.int32)

    def _group(g, carry):
        pltpu.async_copy(z2_hbm.at[gidx_v.at[g]], zbuf, sem).wait()
        for j in range(_G):
            vrow = vals_v[g, pl.ds(j * 16, 16)]
            for cc in range(_C // 16):
                sl = pl.ds(cc * 16, 16)
                v = zbuf[j * _SEQ, sl]
                for sp in range(1, _SEQ):
                    v = v + zbuf[j * _SEQ + sp, sl]
                v = v + bias_v[cc, :]
                v = jnp.where(v > 0.0, v, jnp.exp(v) - 1.0)
                v = v * vrow
                stage[j, sl] = v
        pltpu.async_copy(stage, acc.at[rows_v.at[g]], sem2, add=True).wait()
        return carry

    for p in range(_NP):
        for r in range(10):
            pltpu.sync_copy(zerov, acc.at[pl.ds(sid * 160 + r * 16, 16)])
        plsc.subcore_barrier()
        lax.fori_loop(0, _NG, _group, 0)
        plsc.subcore_barrier()
        pltpu.sync_copy(
            acc.at[pl.ds(sid * 160, 160)],
            out_hbm.at[cid * _NP + p, pl.ds(sid * 160, 160)])
        if p + 1 < _NP:
            def _shift_row(i, carry):
                for c5 in range(_RW // 16):
                    sl = pl.ds(c5 * 16, 16)
                    gidx_v[i, sl] = gidx_v[i, sl] + delta
                return carry

            lax.fori_loop(0, _NG, _shift_row, 0)


_sc_fused = functools.partial(
    pl.kernel,
    out_type=jax.ShapeDtypeStruct((_BS, _MP, _C), jnp.float32),
    mesh=plsc.VectorSubcoreMesh(core_axis_name="c", subcore_axis_name="s",
                                num_cores=_NC, num_subcores=_NS),
    scratch_types=[
        pltpu.VMEM((_NG, _RW), jnp.int32),
        pltpu.VMEM((_NG, 16), jnp.int32),
        pltpu.VMEM((_NG, 128), jnp.float32),
        pltpu.VMEM((1, 16), jnp.int32),
        pltpu.VMEM((_RW, _C), jnp.float32),
        pltpu.VMEM((16, _C), jnp.float32),
        pltpu.VMEM((_C // 16, 16), jnp.float32),
        pltpu.VMEM((16, _C), jnp.float32),
        pltpu.VMEM_SHARED((_MP, _C), jnp.float32),
        pltpu.SemaphoreType.DMA,
        pltpu.SemaphoreType.DMA,
    ],
)(_sc_body)


def kernel(x, indices, dt_rows, dt_cols, dt_vals, W, b):
    idx32 = indices.astype(jnp.int32)
    cols = dt_cols.astype(jnp.int32)
    rows = dt_rows.astype(jnp.int32)

    pad = _NNZP - _NNZ
    cols_p = jnp.pad(cols, (0, pad))
    rows_p = jnp.pad(rows, (0, pad))
    vals_p = jnp.pad(dt_vals, (0, pad))

    nbr = jnp.take(idx32, cols_p, axis=0)
    g9 = nbr * _SEQ + jnp.arange(_SEQ, dtype=jnp.int32)[None, :]
    g72 = g9.reshape(_NS, _NG, _G * _SEQ)
    padcol = (jnp.arange(_NG, dtype=jnp.int32)[:, None] * 8
              + jnp.arange(_RW - _G * _SEQ, dtype=jnp.int32)[None, :]) % (
                  _N * _SEQ)
    gidx16 = jnp.concatenate(
        [g72, jnp.broadcast_to(padcol[None], (_NS, _NG, _RW - _G * _SEQ))],
        axis=2)
    rows16 = jnp.pad(rows_p.reshape(_NS, _NG, _G),
                     ((0, 0), (0, 0), (0, 16 - _G)))
    vals16 = jnp.broadcast_to(vals_p.reshape(_NS, _NG, _G, 1),
                              (_NS, _NG, _G, 16)).reshape(_NS, _NG, 128)

    batoff = jnp.broadcast_to(
        (jnp.arange(_NC, dtype=jnp.int32) * (_NP * _N * _SEQ))[:, None, None],
        (_NC, 1, 16))

    bias_b = b.reshape(_C // 16, 16).astype(jnp.float32)

    wt = W.reshape(_C, _SEQ, _C).transpose(2, 1, 0).reshape(_C, _SEQ * _C)
    z = _matmul_tc(x.reshape(_BS * _N, _C), wt.astype(jnp.float32))
    z2 = z.reshape(_BS * _N * _SEQ, _C)

    pooled_p = _sc_fused(z2, gidx16, rows16, vals16, batoff, bias_b)
    return pooled_p[:, :_M, :]
